# baseline (speedup 1.0000x reference)
import jax
import jax.numpy as jnp
from jax import lax
from jax.experimental import pallas as pl
from jax.experimental.pallas import tpu as pltpu
from jax.experimental.pallas import tpu_sc as plsc

N = 10000
E = 320000
D = 128

NC = 2
NS = 16
NW = NC * NS
CHUNK = 125
E_ROWS = E // CHUNK
N_CHUNKS = E_ROWS // NS
NBUF = 4
SRC_HALF = 80
N_OUTER = N_CHUNKS // NBUF
HALF = 5000
A_PAD = 5248
ROWS_PER_TILE = A_PAD // NS
NTRASH = 128

DEG_CHUNKS = E_ROWS // NW
DEG_PAD = 640
DEG_ROWS = NS * DEG_PAD
DEG_W = 16
DEG_Q = 8

ROW_BLK = 1000
N_BLKS_PER_CORE = HALF // ROW_BLK

_MESH = plsc.VectorSubcoreMesh(core_axis_name="c", subcore_axis_name="s")


def _seg_body(g_hbm, src_hbm, dstt_hbm, zrows_hbm, a_out,
              srcv, dstv, r0buf, r1buf, r2buf, r3buf, acc, gsem, ssem):
    c = lax.axis_index("c")
    s = lax.axis_index("s")
    r0 = s * ROWS_PER_TILE
    rows = [r0buf, r1buf, r2buf, r3buf]
    pltpu.sync_copy(src_hbm.at[pl.ds(s * N_CHUNKS, SRC_HALF)], srcv)
    pltpu.sync_copy(dstt_hbm.at[c, pl.ds(s * N_CHUNKS, SRC_HALF)], dstv)
    pltpu.sync_copy(zrows_hbm, acc.at[pl.ds(r0, ROWS_PER_TILE)])
    plsc.subcore_barrier()

    for b in range(NBUF):
        pltpu.async_copy(g_hbm.at[srcv.at[b]], rows[b], gsem.at[b])

    reload_g = SRC_HALF // NBUF - 1

    def outer(g, carry):
        j0 = g * NBUF
        for b in range(NBUF):
            j = j0 + b
            pltpu.make_async_copy(g_hbm.at[srcv.at[lax.rem(j, SRC_HALF)]],
                                  rows[b], gsem.at[b]).wait()
            pltpu.async_copy(rows[b], acc.at[dstv.at[lax.rem(j, SRC_HALF)]],
                             ssem.at[b], add=True)

        @pl.when(g < N_OUTER - 1)
        def _():
            for b in range(NBUF):
                pltpu.make_async_copy(
                    rows[b], acc.at[dstv.at[lax.rem(j0 + b, SRC_HALF)]],
                    ssem.at[b]).wait()

            @pl.when(g == reload_g)
            def _():
                pltpu.sync_copy(
                    src_hbm.at[pl.ds(s * N_CHUNKS + SRC_HALF, SRC_HALF)],
                    srcv)
                pltpu.sync_copy(
                    dstt_hbm.at[c, pl.ds(s * N_CHUNKS + SRC_HALF, SRC_HALF)],
                    dstv)

            for b in range(NBUF):
                jj = j0 + b + NBUF
                pltpu.async_copy(g_hbm.at[srcv.at[lax.rem(jj, SRC_HALF)]],
                                 rows[b], gsem.at[b])

        return carry

    lax.fori_loop(0, N_OUTER, outer, 0)
    for b in range(NBUF):
        pltpu.make_async_copy(rows[b], acc.at[dstv.at[0]], ssem.at[b]).wait()
    plsc.subcore_barrier()
    pltpu.sync_copy(acc.at[pl.ds(r0, ROWS_PER_TILE)],
                    a_out.at[c, pl.ds(r0, ROWS_PER_TILE)])


_seg = pl.kernel(
    _seg_body,
    mesh=_MESH,
    out_type=jax.ShapeDtypeStruct((NC, A_PAD, D), jnp.float32),
    scratch_types=[
        pltpu.VMEM((SRC_HALF, CHUNK), jnp.int32),
        pltpu.VMEM((SRC_HALF, CHUNK), jnp.int32),
        pltpu.VMEM((CHUNK, D), jnp.float32),
        pltpu.VMEM((CHUNK, D), jnp.float32),
        pltpu.VMEM((CHUNK, D), jnp.float32),
        pltpu.VMEM((CHUNK, D), jnp.float32),
        pltpu.VMEM_SHARED((A_PAD, D), jnp.float32),
        pltpu.SemaphoreType.DMA((NBUF,)),
        pltpu.SemaphoreType.DMA((NBUF,)),
    ],
)


def _dega_body(dstt_hbm, zrows_hbm, ones_hbm, a_out,
               dstv, onesbuf, acc, ssem):
    c = lax.axis_index("c")
    s = lax.axis_index("s")
    r0 = s * ROWS_PER_TILE
    pltpu.sync_copy(dstt_hbm.at[c, pl.ds(s * N_CHUNKS, N_CHUNKS)], dstv)
    pltpu.sync_copy(ones_hbm, onesbuf)
    pltpu.sync_copy(zrows_hbm, acc.at[pl.ds(r0, ROWS_PER_TILE)])
    plsc.subcore_barrier()

    for b in range(2):
        pltpu.async_copy(onesbuf, acc.at[dstv.at[b]], ssem.at[b], add=True)

    def outer(g, carry):
        j0 = g * 2
        for b in range(2):
            j = j0 + b
            pltpu.make_async_copy(onesbuf, acc.at[dstv.at[j]],
                                  ssem.at[b]).wait()

            @pl.when(g < N_CHUNKS // 2 - 1)
            def _():
                pltpu.async_copy(onesbuf, acc.at[dstv.at[j + 2]],
                                 ssem.at[b], add=True)

        return carry

    lax.fori_loop(0, N_CHUNKS // 2, outer, 0)
    plsc.subcore_barrier()
    pltpu.sync_copy(acc.at[pl.ds(r0, ROWS_PER_TILE)],
                    a_out.at[c, pl.ds(r0, ROWS_PER_TILE)])


_dega = pl.kernel(
    _dega_body,
    mesh=_MESH,
    out_type=jax.ShapeDtypeStruct((NC, A_PAD, D), jnp.float32),
    scratch_types=[
        pltpu.VMEM((N_CHUNKS, CHUNK), jnp.int32),
        pltpu.VMEM((CHUNK, D), jnp.float32),
        pltpu.VMEM_SHARED((A_PAD, D), jnp.float32),
        pltpu.SemaphoreType.DMA((2,)),
    ],
)


def _dstt_body(dst_ref, out_ref):
    d = dst_ref[...]
    trash = HALF + (d & (NTRASH - 1))
    out_ref[0] = jnp.where(d < HALF, d, trash)
    out_ref[1] = jnp.where(d >= HALF, d - HALF, trash)


def _dst_transform(dst):
    return pl.pallas_call(
        _dstt_body,
        out_shape=jax.ShapeDtypeStruct((NC, E_ROWS, CHUNK), jnp.int32),
    )(dst)


def _project_body(x_ref, w_ref, b_ref, s_ref, g_ref):
    u = jnp.dot(x_ref[...], w_ref[...], preferred_element_type=jnp.float32)
    s_ref[...] = u[:, :D] + b_ref[...]
    g_ref[...] = u[:, D:]


def _combine_project_body(s_ref, a_ref, dega_ref, w_ref, b_ref,
                          s2_ref, g2_ref):
    scale = 1.0 / jnp.maximum(dega_ref[0], 1.0)
    h = jnp.maximum(s_ref[...] + a_ref[0] * scale, 0.0)
    u = jnp.dot(h, w_ref[...], preferred_element_type=jnp.float32)
    s2_ref[...] = u[:, :D] + b_ref[...]
    g2_ref[...] = u[:, D:]


def _final_body(s_ref, a_ref, dega_ref, o_ref):
    scale = 1.0 / jnp.maximum(dega_ref[0], 1.0)
    o_ref[...] = s_ref[...] + a_ref[0] * scale


_GRID = (N // ROW_BLK,)
_SPEC_ROWS = pl.BlockSpec((ROW_BLK, D), lambda i: (i, 0))
_SPEC_W = pl.BlockSpec((D, 2 * D), lambda i: (0, 0))
_SPEC_B = pl.BlockSpec((1, D), lambda i: (0, 0))
_SPEC_A = pl.BlockSpec(
    (1, ROW_BLK, D),
    lambda i: (i // N_BLKS_PER_CORE, i % N_BLKS_PER_CORE, 0))

_OUT_SG = [jax.ShapeDtypeStruct((N, D), jnp.float32)] * 2


def _project(h, wa, b):
    return pl.pallas_call(
        _project_body,
        grid=_GRID,
        in_specs=[_SPEC_ROWS, _SPEC_W, _SPEC_B],
        out_specs=[_SPEC_ROWS, _SPEC_ROWS],
        out_shape=_OUT_SG,
    )(h, wa, b)


def _combine_project(s, a, dega, wa, b):
    return pl.pallas_call(
        _combine_project_body,
        grid=_GRID,
        in_specs=[_SPEC_ROWS, _SPEC_A, _SPEC_A, _SPEC_W, _SPEC_B],
        out_specs=[_SPEC_ROWS, _SPEC_ROWS],
        out_shape=_OUT_SG,
    )(s, a, dega, wa, b)


def _final(s, a, dega):
    return pl.pallas_call(
        _final_body,
        grid=_GRID,
        in_specs=[_SPEC_ROWS, _SPEC_A, _SPEC_A],
        out_specs=_SPEC_ROWS,
        out_shape=jax.ShapeDtypeStruct((N, D), jnp.float32),
    )(s, a, dega)


def kernel(x, edge_index, W1, b1, W2, b2, W3, b3):
    src = edge_index[0].astype(jnp.int32).reshape(E_ROWS, CHUNK)
    dst = edge_index[1].astype(jnp.int32).reshape(E_ROWS, CHUNK)
    dstt = _dst_transform(dst)

    def wcat(w):
        return jnp.concatenate([w[:, :D].T, w[:, D:].T], axis=1)

    zrows = jnp.zeros((ROWS_PER_TILE, D), jnp.float32)

    ones = jnp.ones((CHUNK, D), jnp.float32)
    dega = _dega(dstt, zrows, ones)
    s1, g1 = _project(x, wcat(W1), b1.reshape(1, D))
    a1 = _seg(g1, src, dstt, zrows)
    s2, g2 = _combine_project(s1, a1, dega, wcat(W2), b2.reshape(1, D))
    a2 = _seg(g2, src, dstt, zrows)
    s3, g3 = _combine_project(s2, a2, dega, wcat(W3), b3.reshape(1, D))
    a3 = _seg(g3, src, dstt, zrows)
    return _final(s3, a3, dega)


# speedup vs baseline: 1.5848x; 1.5848x over previous
import jax
import jax.numpy as jnp
from jax import lax
from jax.experimental import pallas as pl
from jax.experimental.pallas import tpu as pltpu
from jax.experimental.pallas import tpu_sc as plsc

N = 10000
E = 320000
D = 128

NC = 2
NS = 16
NW = NC * NS
CHUNK = 125
E_ROWS = E // CHUNK
N_CHUNKS = E_ROWS // NW
NBUF = 2
N_OUTER = N_CHUNKS // NBUF
SRC_HALF = N_CHUNKS // 2
N_PAD = 10240
ROWS_PER_TILE = N_PAD // NS

ROW_BLK = 1000

_MESH = plsc.VectorSubcoreMesh(core_axis_name="c", subcore_axis_name="s")


def _seg_body(g_hbm, src_hbm, dst_hbm, zrows_hbm, a_out,
              srcv, dstv, r0buf, r1buf, acc, gsem, ssem):
    c = lax.axis_index("c")
    s = lax.axis_index("s")
    wid = c * NS + s
    e0 = wid * N_CHUNKS
    r0 = s * ROWS_PER_TILE
    rows = [r0buf, r1buf]
    pltpu.sync_copy(src_hbm.at[pl.ds(e0, SRC_HALF)], srcv)
    pltpu.sync_copy(dst_hbm.at[pl.ds(e0, SRC_HALF)], dstv)
    pltpu.sync_copy(zrows_hbm, acc.at[pl.ds(r0, ROWS_PER_TILE)])
    plsc.subcore_barrier()

    for b in range(NBUF):
        pltpu.async_copy(g_hbm.at[srcv.at[b]], rows[b], gsem.at[b])

    reload_g = SRC_HALF // NBUF - 1

    def outer(g, carry):
        j0 = g * NBUF
        for b in range(NBUF):
            j = j0 + b
            pltpu.make_async_copy(g_hbm.at[srcv.at[lax.rem(j, SRC_HALF)]],
                                  rows[b], gsem.at[b]).wait()
            pltpu.async_copy(rows[b], acc.at[dstv.at[lax.rem(j, SRC_HALF)]],
                             ssem.at[b], add=True)

        @pl.when(g < N_OUTER - 1)
        def _():
            for b in range(NBUF):
                pltpu.make_async_copy(
                    rows[b], acc.at[dstv.at[lax.rem(j0 + b, SRC_HALF)]],
                    ssem.at[b]).wait()

            @pl.when(g == reload_g)
            def _():
                pltpu.sync_copy(src_hbm.at[pl.ds(e0 + SRC_HALF, SRC_HALF)],
                                srcv)
                pltpu.sync_copy(dst_hbm.at[pl.ds(e0 + SRC_HALF, SRC_HALF)],
                                dstv)

            for b in range(NBUF):
                jj = j0 + b + NBUF
                pltpu.async_copy(g_hbm.at[srcv.at[lax.rem(jj, SRC_HALF)]],
                                 rows[b], gsem.at[b])

        return carry

    lax.fori_loop(0, N_OUTER, outer, 0)
    for b in range(NBUF):
        pltpu.make_async_copy(rows[b], acc.at[dstv.at[0]], ssem.at[b]).wait()
    plsc.subcore_barrier()
    pltpu.sync_copy(acc.at[pl.ds(r0, ROWS_PER_TILE)],
                    a_out.at[c, pl.ds(r0, ROWS_PER_TILE)])


_seg = pl.kernel(
    _seg_body,
    mesh=_MESH,
    out_type=jax.ShapeDtypeStruct((NC, N_PAD, D), jnp.float32),
    scratch_types=[
        pltpu.VMEM((SRC_HALF, CHUNK), jnp.int32),
        pltpu.VMEM((SRC_HALF, CHUNK), jnp.int32),
        pltpu.VMEM((CHUNK, D), jnp.float32),
        pltpu.VMEM((CHUNK, D), jnp.float32),
        pltpu.VMEM_SHARED((N_PAD, D), jnp.float32),
        pltpu.SemaphoreType.DMA((NBUF,)),
        pltpu.SemaphoreType.DMA((NBUF,)),
    ],
)


def _dega_body(dst_hbm, zrows_hbm, ones_hbm, a_out,
               dstv, onesbuf, acc, ssem):
    c = lax.axis_index("c")
    s = lax.axis_index("s")
    wid = c * NS + s
    e0 = wid * N_CHUNKS
    r0 = s * ROWS_PER_TILE
    pltpu.sync_copy(dst_hbm.at[pl.ds(e0, N_CHUNKS)], dstv)
    pltpu.sync_copy(ones_hbm, onesbuf)
    pltpu.sync_copy(zrows_hbm, acc.at[pl.ds(r0, ROWS_PER_TILE)])
    plsc.subcore_barrier()

    for b in range(2):
        pltpu.async_copy(onesbuf, acc.at[dstv.at[b]], ssem.at[b], add=True)

    def outer(g, carry):
        j0 = g * 2
        for b in range(2):
            j = j0 + b
            pltpu.make_async_copy(onesbuf, acc.at[dstv.at[j]],
                                  ssem.at[b]).wait()

            @pl.when(g < N_CHUNKS // 2 - 1)
            def _():
                pltpu.async_copy(onesbuf, acc.at[dstv.at[j + 2]],
                                 ssem.at[b], add=True)

        return carry

    lax.fori_loop(0, N_CHUNKS // 2, outer, 0)
    plsc.subcore_barrier()
    pltpu.sync_copy(acc.at[pl.ds(r0, ROWS_PER_TILE)],
                    a_out.at[c, pl.ds(r0, ROWS_PER_TILE)])


_dega = pl.kernel(
    _dega_body,
    mesh=_MESH,
    out_type=jax.ShapeDtypeStruct((NC, N_PAD, D), jnp.float32),
    scratch_types=[
        pltpu.VMEM((N_CHUNKS, CHUNK), jnp.int32),
        pltpu.VMEM((CHUNK, D), jnp.float32),
        pltpu.VMEM_SHARED((N_PAD, D), jnp.float32),
        pltpu.SemaphoreType.DMA((2,)),
    ],
)


def _project_body(x_ref, w_ref, b_ref, s_ref, g_ref):
    u = jnp.dot(x_ref[...], w_ref[...], preferred_element_type=jnp.float32)
    s_ref[...] = u[:, :D] + b_ref[...]
    g_ref[...] = u[:, D:]


def _combine_project_body(s_ref, a_ref, dega_ref, w_ref, b_ref,
                          s2_ref, g2_ref):
    deg = dega_ref[0, :, 0] + dega_ref[1, :, 0]
    scale = 1.0 / jnp.maximum(deg, 1.0)
    agg = a_ref[0] + a_ref[1]
    h = jnp.maximum(s_ref[...] + agg * scale[:, None], 0.0)
    u = jnp.dot(h, w_ref[...], preferred_element_type=jnp.float32)
    s2_ref[...] = u[:, :D] + b_ref[...]
    g2_ref[...] = u[:, D:]


def _final_body(s_ref, a_ref, dega_ref, o_ref):
    deg = dega_ref[0, :, 0] + dega_ref[1, :, 0]
    scale = 1.0 / jnp.maximum(deg, 1.0)
    o_ref[...] = s_ref[...] + (a_ref[0] + a_ref[1]) * scale[:, None]


_GRID = (N // ROW_BLK,)
_SPEC_ROWS = pl.BlockSpec((ROW_BLK, D), lambda i: (i, 0))
_SPEC_W = pl.BlockSpec((D, 2 * D), lambda i: (0, 0))
_SPEC_B = pl.BlockSpec((1, D), lambda i: (0, 0))
_SPEC_A = pl.BlockSpec((NC, ROW_BLK, D), lambda i: (0, i, 0))

_OUT_SG = [jax.ShapeDtypeStruct((N, D), jnp.float32)] * 2


def _project(h, wa, b):
    return pl.pallas_call(
        _project_body,
        grid=_GRID,
        in_specs=[_SPEC_ROWS, _SPEC_W, _SPEC_B],
        out_specs=[_SPEC_ROWS, _SPEC_ROWS],
        out_shape=_OUT_SG,
    )(h, wa, b)


def _combine_project(s, a, dega, wa, b):
    return pl.pallas_call(
        _combine_project_body,
        grid=_GRID,
        in_specs=[_SPEC_ROWS, _SPEC_A, _SPEC_A, _SPEC_W, _SPEC_B],
        out_specs=[_SPEC_ROWS, _SPEC_ROWS],
        out_shape=_OUT_SG,
    )(s, a, dega, wa, b)


def _final(s, a, dega):
    return pl.pallas_call(
        _final_body,
        grid=_GRID,
        in_specs=[_SPEC_ROWS, _SPEC_A, _SPEC_A],
        out_specs=_SPEC_ROWS,
        out_shape=jax.ShapeDtypeStruct((N, D), jnp.float32),
    )(s, a, dega)


def kernel(x, edge_index, W1, b1, W2, b2, W3, b3):
    src = edge_index[0].astype(jnp.int32).reshape(E_ROWS, CHUNK)
    dst = edge_index[1].astype(jnp.int32).reshape(E_ROWS, CHUNK)

    def wcat(w):
        return jnp.concatenate([w[:, :D].T, w[:, D:].T], axis=1)

    zrows = jnp.zeros((ROWS_PER_TILE, D), jnp.float32)
    ones = jnp.ones((CHUNK, D), jnp.float32)

    dega = _dega(dst, zrows, ones)
    s1, g1 = _project(x, wcat(W1), b1.reshape(1, D))
    a1 = _seg(g1, src, dst, zrows)
    s2, g2 = _combine_project(s1, a1, dega, wcat(W2), b2.reshape(1, D))
    a2 = _seg(g2, src, dst, zrows)
    s3, g3 = _combine_project(s2, a2, dega, wcat(W3), b3.reshape(1, D))
    a3 = _seg(g3, src, dst, zrows)
    return _final(s3, a3, dega)
